# baseline (speedup 1.0000x reference)
import functools

import jax
import jax.numpy as jnp
from jax.experimental import pallas as pl
from jax.experimental.pallas import tpu as pltpu


def _plane_sum_kernel(x_ref, o_ref, *, inv_count):
    s = jnp.sum(x_ref[...].astype(jnp.float32), axis=0)
    o_ref[...] = (s * inv_count).astype(o_ref.dtype)


def kernel(x):
    N, C, H, W = x.shape
    K = H * W
    if x.size == 0:
        return jnp.zeros((N, C), dtype=x.dtype)

    xp = x.transpose(2, 3, 0, 1).reshape(K, N, C)

    bn = 16 if N % 16 == 0 else N
    return pl.pallas_call(
        functools.partial(_plane_sum_kernel, inv_count=1.0 / float(K)),
        out_shape=jax.ShapeDtypeStruct((N, C), x.dtype),
        grid=(N // bn,),
        in_specs=[pl.BlockSpec((K, bn, C), lambda i: (0, i, 0))],
        out_specs=pl.BlockSpec((bn, C), lambda i: (i, 0)),
        compiler_params=pltpu.CompilerParams(
            dimension_semantics=("parallel",),
        ),
    )(xp)


# speedup vs baseline: 1.6186x; 1.6186x over previous
import functools

import jax
import jax.numpy as jnp
from jax.experimental import pallas as pl
from jax.experimental.pallas import tpu as pltpu


def _plane_sum_kernel(x_ref, o_ref, *, inv_count):
    s = jnp.sum(x_ref[...].astype(jnp.float32), axis=0)
    o_ref[...] = (s * inv_count).astype(o_ref.dtype)


def kernel(x):
    N, C, H, W = x.shape
    K = H * W
    if x.size == 0:
        return jnp.zeros((N, C), dtype=x.dtype)

    xp = x.transpose(2, 3, 0, 1).reshape(K, N, C)

    bn = 64 if N % 64 == 0 else N
    return pl.pallas_call(
        functools.partial(_plane_sum_kernel, inv_count=1.0 / float(K)),
        out_shape=jax.ShapeDtypeStruct((N, C), x.dtype),
        grid=(N // bn,),
        in_specs=[pl.BlockSpec((K, bn, C), lambda i: (0, i, 0))],
        out_specs=pl.BlockSpec((bn, C), lambda i: (i, 0)),
        compiler_params=pltpu.CompilerParams(
            dimension_semantics=("parallel",),
        ),
    )(xp)
